# baseline (speedup 1.0000x reference)
import functools

import jax
import jax.numpy as jnp
from jax import lax
from jax.experimental import pallas as pl
from jax.experimental.pallas import tpu as pltpu
from jax.experimental.pallas import tpu_sc as plsc

NC = 2
NS = 16
NW = NC * NS
L = 16


def _emb_gather(weight, idx_t, B, S, D):
    V, _ = weight.shape
    bpw = B // NW
    dg = D // 8
    mesh = plsc.VectorSubcoreMesh(core_axis_name="c", subcore_axis_name="s")

    @functools.partial(
        pl.kernel,
        out_type=jax.ShapeDtypeStruct((S, dg, NW, 8, bpw), jnp.float32),
        mesh=mesh,
        compiler_params=pltpu.CompilerParams(
            use_tc_tiling_on_sc=False, needs_layout_passes=False),
        scratch_types=[
            pltpu.VMEM((S, bpw), jnp.int32),
            pltpu.VMEM((bpw, D), jnp.float32),
            pltpu.VMEM((bpw, D), jnp.float32),
            pltpu.VMEM((dg, 8, bpw), jnp.float32),
            pltpu.VMEM((dg, 8, bpw), jnp.float32),
            pltpu.SemaphoreType.DMA,
            pltpu.SemaphoreType.DMA,
            pltpu.SemaphoreType.DMA,
            pltpu.SemaphoreType.DMA,
        ],
    )
    def k(w_hbm, idx_hbm, out_hbm, idx_v, buf_a, buf_b, tbuf_a, tbuf_b,
          gsem_a, gsem_b, wsem_a, wsem_b):
        wid = lax.axis_index("s") * NC + lax.axis_index("c")
        pltpu.sync_copy(idx_hbm.at[:, pl.ds(wid * bpw, bpw)], idx_v)

        banks = ((buf_a, tbuf_a, gsem_a, wsem_a),
                 (buf_b, tbuf_b, gsem_b, wsem_b))
        lanes = lax.broadcasted_iota(jnp.int32, (L,), 0)

        def gather_start(s, p):
            buf, _, gsem, _ = banks[p]
            pltpu.async_copy(w_hbm.at[idx_v.at[s]], buf, gsem)

        def gather_wait(s, p):
            buf, _, gsem, _ = banks[p]
            pltpu.make_async_copy(w_hbm.at[idx_v.at[s]], buf, gsem).wait()

        def transpose(p):
            buf, tbuf, _, _ = banks[p]
            for j in range(D):
                for kk in range(bpw // L):
                    col = lanes * 0 + j
                    vec = plsc.load_gather(buf, [lanes + kk * L, col])
                    tbuf[j // 8, j % 8, pl.ds(kk * L, L)] = vec

        def write_start(s, p):
            tbuf, wsem = banks[p][1], banks[p][3]
            pltpu.async_copy(tbuf, out_hbm.at[s, :, wid], wsem)

        def write_wait(s, p):
            tbuf, wsem = banks[p][1], banks[p][3]
            pltpu.make_async_copy(tbuf, out_hbm.at[s, :, wid], wsem).wait()

        gather_start(0, 0)

        @pl.loop(0, S, step=2)
        def _(g):
            for p in range(2):
                s = g + p

                @pl.when(s + 1 < S)
                def _():
                    @pl.when(s >= 1)
                    def _():
                        write_wait(s - 1, 1 - p)
                    gather_start(s + 1, 1 - p)

                gather_wait(s, p)
                transpose(p)
                write_start(s, p)

        write_wait(S - 2, 0)
        write_wait(S - 1, 1)

    return k(weight, idx_t)


def kernel(x, weight):
    B, S = x.shape
    V, D = weight.shape
    idx_t = x.T.astype(jnp.int32)
    out5 = _emb_gather(weight, idx_t, B, S, D)
    return out5.transpose((2, 4, 0, 1, 3)).reshape(B, S, D)


# speedup vs baseline: 2.2227x; 2.2227x over previous
import functools

import jax
import jax.numpy as jnp
from jax import lax
from jax.experimental import pallas as pl
from jax.experimental.pallas import tpu as pltpu
from jax.experimental.pallas import tpu_sc as plsc

NC = 2
NS = 16
NW = NC * NS
K = 1


def _emb_gather(weight, idx3, D):
    B, nh, ch = idx3.shape
    S = nh * ch
    V, _ = weight.shape
    DP = 2 * D
    rows_per_w = B // NW
    ngroups = rows_per_w // K
    mesh = plsc.VectorSubcoreMesh(core_axis_name="c", subcore_axis_name="s")

    @functools.partial(
        pl.kernel,
        out_type=jax.ShapeDtypeStruct((B, S, DP), jnp.float32),
        mesh=mesh,
        compiler_params=pltpu.CompilerParams(use_tc_tiling_on_sc=False),
        scratch_types=[
            pltpu.VMEM((rows_per_w, nh, ch), jnp.int32),
            pltpu.VMEM((K, S, D), jnp.float32),
            pltpu.VMEM((K, S, D), jnp.float32),
            pltpu.SemaphoreType.DMA,
            pltpu.SemaphoreType.DMA,
            pltpu.SemaphoreType.DMA,
            pltpu.SemaphoreType.DMA,
        ],
    )
    def k(w_hbm, idx_hbm, out_hbm, idx_v, rows_a, rows_b,
          gsem_a, gsem_b, wsem_a, wsem_b):
        wid = lax.axis_index("s") * NC + lax.axis_index("c")
        row0 = wid * rows_per_w
        w_rows = w_hbm
        pltpu.sync_copy(idx_hbm.at[pl.ds(row0, rows_per_w)], idx_v)

        banks = ((rows_a, gsem_a, wsem_a), (rows_b, gsem_b, wsem_b))

        def gather_start(bank, local_row, b):
            rows, gsem, _ = banks[bank]
            for h in range(nh):
                pltpu.async_copy(
                    w_rows.at[idx_v.at[local_row, h]],
                    rows.at[b, pl.ds(h * ch, ch)], gsem)

        def write_dst(local_row):
            return out_hbm.at[row0 + local_row, :, pl.ds(0, D)]

        def gather_wait(bank, local_row, b):
            rows, gsem, _ = banks[bank]
            for h in range(nh):
                pltpu.make_async_copy(
                    w_rows.at[idx_v.at[local_row, h]],
                    rows.at[b, pl.ds(h * ch, ch)], gsem).wait()

        def write_start(bank, local_row, b):
            rows, _, wsem = banks[bank]
            pltpu.async_copy(rows.at[b], write_dst(local_row), wsem)

        def write_wait(bank, local_row, b):
            rows, _, wsem = banks[bank]
            pltpu.make_async_copy(rows.at[b], write_dst(local_row), wsem).wait()

        for b in range(K):
            gather_start(0, b, b)

        @pl.loop(0, ngroups, step=2)
        def _(g):
            for p in range(2):
                gp = g + p

                @pl.when(gp + 1 < ngroups)
                def _():
                    @pl.when(gp >= 1)
                    def _():
                        for b in range(K):
                            write_wait(1 - p, (gp - 1) * K + b, b)
                    for b in range(K):
                        gather_start(1 - p, (gp + 1) * K + b, b)

                for b in range(K):
                    gather_wait(p, gp * K + b, b)
                for b in range(K):
                    write_start(p, gp * K + b, b)

        for b in range(K):
            write_wait((ngroups - 1) % 2, (ngroups - 1) * K + b, b)

    return k(weight, idx3)


def kernel(x, weight):
    B, S = x.shape
    V, D = weight.shape
    idx3 = x.reshape(B, 2, S // 2).astype(jnp.int32)
    out128 = _emb_gather(weight, idx3, D)
    return out128[:, :, :D]
